# Initial kernel scaffold; baseline (speedup 1.0000x reference)
import jax
import jax.numpy as jnp
from jax.experimental import pallas as pl


def kernel(x, position_ids, cos_cached, sin_cached):
    raise NotImplementedError("write your pallas kernel here")


# speedup vs baseline: 4.8618x; 4.8618x over previous
import functools

import jax
import jax.numpy as jnp
from jax import lax
from jax.experimental import pallas as pl
from jax.experimental.pallas import tpu as pltpu
from jax.experimental.pallas import tpu_sc as plsc

DIM = 128
CHUNK = 128


def _build_sc_gather(n_tokens: int):
    info = plsc.get_sparse_core_info()
    nc, ns = info.num_cores, info.num_subcores
    nw = nc * ns
    b_per_w = n_tokens // nw
    assert n_tokens % nw == 0 and b_per_w % CHUNK == 0
    n_chunks = b_per_w // CHUNK

    mesh = plsc.VectorSubcoreMesh(core_axis_name="c", subcore_axis_name="s")
    out = jax.ShapeDtypeStruct((n_tokens, DIM), jnp.float32)

    @functools.partial(
        pl.kernel,
        mesh=mesh,
        out_type=(out, out),
        scratch_types=[
            pltpu.VMEM((n_chunks, CHUNK), jnp.int32),
            pltpu.VMEM((CHUNK, DIM), jnp.float32),
            pltpu.VMEM((CHUNK, DIM), jnp.float32),
            pltpu.SemaphoreType.DMA,
            pltpu.SemaphoreType.DMA,
        ],
    )
    def gather_kernel(pos_hbm, cos_hbm, sin_hbm, cos_out, sin_out,
                      idx_v, cos_rows, sin_rows, sem_c, sem_s):
        wid = lax.axis_index("s") * nc + lax.axis_index("c")
        pltpu.sync_copy(pos_hbm.at[pl.ds(wid * n_chunks, n_chunks)], idx_v)
        for c in range(n_chunks):
            row = idx_v.at[c]
            cp_c = pltpu.async_copy(cos_hbm.at[row], cos_rows, sem_c)
            cp_s = pltpu.async_copy(sin_hbm.at[row], sin_rows, sem_s)
            base = wid * b_per_w + c * CHUNK
            cp_c.wait()
            pltpu.sync_copy(cos_rows, cos_out.at[pl.ds(base, CHUNK)])
            cp_s.wait()
            pltpu.sync_copy(sin_rows, sin_out.at[pl.ds(base, CHUNK)])

    return gather_kernel


def kernel(x, position_ids, cos_cached, sin_cached):
    b, s = position_ids.shape
    n_tokens = b * s
    pos2d = position_ids.astype(jnp.int32).reshape(n_tokens // CHUNK, CHUNK)
    gather = _build_sc_gather(n_tokens)
    cos_flat, sin_flat = gather(pos2d, cos_cached, sin_cached)
    cos = cos_flat.reshape(b, s, DIM).astype(x.dtype)
    sin = sin_flat.reshape(b, s, DIM).astype(x.dtype)
    return (cos, sin)


# speedup vs baseline: 5.2238x; 1.0745x over previous
import functools

import jax
import jax.numpy as jnp
from jax import lax
from jax.experimental import pallas as pl
from jax.experimental.pallas import tpu as pltpu
from jax.experimental.pallas import tpu_sc as plsc

DIM = 128
CHUNK = 128


def _build_sc_gather(n_tokens: int):
    info = plsc.get_sparse_core_info()
    nc, ns = info.num_cores, info.num_subcores
    nw = nc * ns
    b_per_w = n_tokens // nw
    assert n_tokens % nw == 0 and b_per_w % CHUNK == 0
    n_chunks = b_per_w // CHUNK

    mesh = plsc.VectorSubcoreMesh(core_axis_name="c", subcore_axis_name="s")
    out = jax.ShapeDtypeStruct((n_tokens, DIM), jnp.float32)

    @functools.partial(
        pl.kernel,
        mesh=mesh,
        out_type=(out, out),
        scratch_types=[
            pltpu.VMEM((n_chunks, CHUNK), jnp.int32),
            pltpu.VMEM((2, CHUNK, DIM), jnp.float32),
            pltpu.VMEM((2, CHUNK, DIM), jnp.float32),
            pltpu.SemaphoreType.DMA,
            pltpu.SemaphoreType.DMA,
            pltpu.SemaphoreType.DMA,
            pltpu.SemaphoreType.DMA,
        ],
    )
    def gather_kernel(pos_hbm, cos_hbm, sin_hbm, cos_out, sin_out,
                      idx_v, cos_rows, sin_rows, sem_gc, sem_gs,
                      sem_wc, sem_ws):
        wid = lax.axis_index("s") * nc + lax.axis_index("c")
        pltpu.sync_copy(pos_hbm.at[pl.ds(wid * n_chunks, n_chunks)], idx_v)

        def issue_gather(c):
            b = c % 2
            return (
                pltpu.async_copy(cos_hbm.at[idx_v.at[c]], cos_rows.at[b], sem_gc),
                pltpu.async_copy(sin_hbm.at[idx_v.at[c]], sin_rows.at[b], sem_gs),
            )

        def issue_write(c):
            b = c % 2
            dst = pl.ds(wid * b_per_w + c * CHUNK, CHUNK)
            return (
                pltpu.async_copy(cos_rows.at[b], cos_out.at[dst], sem_wc),
                pltpu.async_copy(sin_rows.at[b], sin_out.at[dst], sem_ws),
            )

        gathers = {0: issue_gather(0)}
        writes = {}
        for c in range(n_chunks):
            if c + 1 < n_chunks:
                if c >= 1:
                    for op in writes.pop(c - 1):
                        op.wait()
                gathers[c + 1] = issue_gather(c + 1)
            for op in gathers.pop(c):
                op.wait()
            writes[c] = issue_write(c)
        for c in (n_chunks - 2, n_chunks - 1):
            for op in writes.pop(c):
                op.wait()

    return gather_kernel


def kernel(x, position_ids, cos_cached, sin_cached):
    b, s = position_ids.shape
    n_tokens = b * s
    pos2d = position_ids.astype(jnp.int32).reshape(n_tokens // CHUNK, CHUNK)
    gather = _build_sc_gather(n_tokens)
    cos_flat, sin_flat = gather(pos2d, cos_cached, sin_cached)
    cos = cos_flat.reshape(b, s, DIM).astype(x.dtype)
    sin = sin_flat.reshape(b, s, DIM).astype(x.dtype)
    return (cos, sin)


# speedup vs baseline: 5.7942x; 1.1092x over previous
import functools

import jax
import jax.numpy as jnp
from jax import lax
from jax.experimental import pallas as pl
from jax.experimental.pallas import tpu as pltpu
from jax.experimental.pallas import tpu_sc as plsc

DIM = 128
HALF = 64
CHUNK = 128


def _build_sc_gather(n_tokens: int):
    info = plsc.get_sparse_core_info()
    nc, ns = info.num_cores, info.num_subcores
    nw = nc * ns
    b_per_w = n_tokens // nw
    assert n_tokens % nw == 0 and b_per_w % CHUNK == 0
    n_chunks = b_per_w // CHUNK

    mesh = plsc.VectorSubcoreMesh(core_axis_name="c", subcore_axis_name="s")
    out = jax.ShapeDtypeStruct((2 * n_tokens, HALF), jnp.float32)

    @functools.partial(
        pl.kernel,
        mesh=mesh,
        out_type=(out, out),
        compiler_params=pltpu.CompilerParams(use_tc_tiling_on_sc=False),
        scratch_types=[
            pltpu.VMEM((n_chunks, CHUNK), jnp.int32),
            pltpu.VMEM((n_chunks, CHUNK), jnp.int32),
            pltpu.VMEM((n_chunks, CHUNK), jnp.int32),
            pltpu.VMEM((2, CHUNK, HALF), jnp.float32),
            pltpu.VMEM((2, CHUNK, HALF), jnp.float32),
            pltpu.SemaphoreType.DMA,
            pltpu.SemaphoreType.DMA,
            pltpu.SemaphoreType.DMA,
            pltpu.SemaphoreType.DMA,
        ],
    )
    def gather_kernel(gidx_hbm, weven_hbm, wodd_hbm, cos_hbm, sin_hbm,
                      cos_out, sin_out,
                      gidx_v, weven_v, wodd_v, cos_rows, sin_rows,
                      sem_gc, sem_gs, sem_wc, sem_ws):
        wid = lax.axis_index("s") * nc + lax.axis_index("c")
        slab = pl.ds(wid * n_chunks, n_chunks)
        pltpu.sync_copy(gidx_hbm.at[slab], gidx_v)
        pltpu.sync_copy(weven_hbm.at[slab], weven_v)
        pltpu.sync_copy(wodd_hbm.at[slab], wodd_v)

        def issue_gather(c):
            b = c % 2
            return (
                pltpu.async_copy(cos_hbm.at[gidx_v.at[c]], cos_rows.at[b], sem_gc),
                pltpu.async_copy(sin_hbm.at[gidx_v.at[c]], sin_rows.at[b], sem_gs),
            )

        def issue_write(c):
            b = c % 2
            return (
                pltpu.async_copy(cos_rows.at[b], cos_out.at[weven_v.at[c]], sem_wc),
                pltpu.async_copy(cos_rows.at[b], cos_out.at[wodd_v.at[c]], sem_wc),
                pltpu.async_copy(sin_rows.at[b], sin_out.at[weven_v.at[c]], sem_ws),
                pltpu.async_copy(sin_rows.at[b], sin_out.at[wodd_v.at[c]], sem_ws),
            )

        gathers = {0: issue_gather(0)}
        writes = {}
        for c in range(n_chunks):
            if c + 1 < n_chunks:
                if c >= 1:
                    for op in writes.pop(c - 1):
                        op.wait()
                gathers[c + 1] = issue_gather(c + 1)
            for op in gathers.pop(c):
                op.wait()
            writes[c] = issue_write(c)
        for c in (n_chunks - 2, n_chunks - 1):
            for op in writes.pop(c):
                op.wait()

    return gather_kernel


def kernel(x, position_ids, cos_cached, sin_cached):
    b, s = position_ids.shape
    n_tokens = b * s
    shape2d = (n_tokens // CHUNK, CHUNK)
    gidx = (position_ids.astype(jnp.int32) * 2).reshape(shape2d)
    tok2 = 2 * jnp.arange(n_tokens, dtype=jnp.int32)
    weven = tok2.reshape(shape2d)
    wodd = (tok2 + 1).reshape(shape2d)
    cos_half = cos_cached.reshape(-1, HALF)
    sin_half = sin_cached.reshape(-1, HALF)
    gather = _build_sc_gather(n_tokens)
    cos_flat, sin_flat = gather(gidx, weven, wodd, cos_half, sin_half)
    cos = cos_flat.reshape(b, s, DIM).astype(x.dtype)
    sin = sin_flat.reshape(b, s, DIM).astype(x.dtype)
    return (cos, sin)


# speedup vs baseline: 6.1108x; 1.0546x over previous
import functools

import jax
import jax.numpy as jnp
from jax import lax
from jax.experimental import pallas as pl
from jax.experimental.pallas import tpu as pltpu
from jax.experimental.pallas import tpu_sc as plsc

DIM = 128
HALF = 64
CHUNK = 256


def _build_sc_gather(n_tokens: int):
    info = plsc.get_sparse_core_info()
    nc, ns = info.num_cores, info.num_subcores
    nw = nc * ns
    b_per_w = n_tokens // nw
    assert n_tokens % nw == 0 and b_per_w % CHUNK == 0
    n_chunks = b_per_w // CHUNK

    mesh = plsc.VectorSubcoreMesh(core_axis_name="c", subcore_axis_name="s")
    out = jax.ShapeDtypeStruct((2 * n_tokens, HALF), jnp.float32)

    @functools.partial(
        pl.kernel,
        mesh=mesh,
        out_type=(out, out),
        compiler_params=pltpu.CompilerParams(use_tc_tiling_on_sc=False),
        scratch_types=[
            pltpu.VMEM((n_chunks, CHUNK), jnp.int32),
            pltpu.VMEM((n_chunks, CHUNK), jnp.int32),
            pltpu.VMEM((n_chunks, CHUNK), jnp.int32),
            pltpu.VMEM((2, CHUNK, HALF), jnp.float32),
            pltpu.VMEM((2, CHUNK, HALF), jnp.float32),
            pltpu.SemaphoreType.DMA,
            pltpu.SemaphoreType.DMA,
            pltpu.SemaphoreType.DMA,
            pltpu.SemaphoreType.DMA,
        ],
    )
    def gather_kernel(gidx_hbm, weven_hbm, wodd_hbm, cos_hbm, sin_hbm,
                      cos_out, sin_out,
                      gidx_v, weven_v, wodd_v, cos_rows, sin_rows,
                      sem_gc, sem_gs, sem_wc, sem_ws):
        wid = lax.axis_index("s") * nc + lax.axis_index("c")
        slab = pl.ds(wid * n_chunks, n_chunks)
        pltpu.sync_copy(gidx_hbm.at[slab], gidx_v)
        pltpu.sync_copy(weven_hbm.at[slab], weven_v)
        pltpu.sync_copy(wodd_hbm.at[slab], wodd_v)

        def issue_gather(c):
            b = c % 2
            return (
                pltpu.async_copy(cos_hbm.at[gidx_v.at[c]], cos_rows.at[b], sem_gc),
                pltpu.async_copy(sin_hbm.at[gidx_v.at[c]], sin_rows.at[b], sem_gs),
            )

        def issue_write(c):
            b = c % 2
            return (
                pltpu.async_copy(cos_rows.at[b], cos_out.at[weven_v.at[c]], sem_wc),
                pltpu.async_copy(cos_rows.at[b], cos_out.at[wodd_v.at[c]], sem_wc),
                pltpu.async_copy(sin_rows.at[b], sin_out.at[weven_v.at[c]], sem_ws),
                pltpu.async_copy(sin_rows.at[b], sin_out.at[wodd_v.at[c]], sem_ws),
            )

        gathers = {0: issue_gather(0)}
        writes = {}
        for c in range(n_chunks):
            if c + 1 < n_chunks:
                if c >= 1:
                    for op in writes.pop(c - 1):
                        op.wait()
                gathers[c + 1] = issue_gather(c + 1)
            for op in gathers.pop(c):
                op.wait()
            writes[c] = issue_write(c)
        for c in (n_chunks - 2, n_chunks - 1):
            for op in writes.pop(c):
                op.wait()

    return gather_kernel


def kernel(x, position_ids, cos_cached, sin_cached):
    b, s = position_ids.shape
    n_tokens = b * s
    shape2d = (n_tokens // CHUNK, CHUNK)
    gidx = (position_ids.astype(jnp.int32) * 2).reshape(shape2d)
    tok2 = 2 * jnp.arange(n_tokens, dtype=jnp.int32)
    weven = tok2.reshape(shape2d)
    wodd = (tok2 + 1).reshape(shape2d)
    cos_half = cos_cached.reshape(-1, HALF)
    sin_half = sin_cached.reshape(-1, HALF)
    gather = _build_sc_gather(n_tokens)
    cos_flat, sin_flat = gather(gidx, weven, wodd, cos_half, sin_half)
    cos = cos_flat.reshape(b, s, DIM).astype(x.dtype)
    sin = sin_flat.reshape(b, s, DIM).astype(x.dtype)
    return (cos, sin)
